# baseline (speedup 1.0000x reference)
import jax
import jax.numpy as jnp
from jax import lax
from jax.experimental import pallas as pl
from jax.experimental.pallas import tpu as pltpu
from jax.experimental.pallas import tpu_sc as plsc

_B = 4096
_L = 50
_V = 100000
_D = 64
_NC = 2
_NS = 16
_NW = _NC * _NS
_RPW = _B // _NW
_NG = _RPW // 16
_VB = 16384
_NVB = 7
_TR = _NVB * _VB // 128


def _tc_matvec_body(xt_ref, w_ref, o_ref):
    xt = xt_ref[...]
    w = w_ref[...]
    o = lax.dot_general(w, xt, (((1,), (0,)), ((), ())),
                        preferred_element_type=jnp.float32)
    o_ref[...] = o.astype(jnp.bfloat16).reshape(_VB // 128, 128)


def _tc_matvec(emb_t, W):
    return pl.pallas_call(
        _tc_matvec_body,
        grid=(_NVB,),
        in_specs=[
            pl.BlockSpec((_D, _VB), lambda i: (0, i)),
            pl.BlockSpec((1, _D), lambda i: (0, 0)),
        ],
        out_specs=pl.BlockSpec((_VB // 128, 128), lambda i: (i, 0)),
        out_shape=jax.ShapeDtypeStruct((_TR, 128), jnp.bfloat16),
    )(emb_t, W)


def _sc_pool_body(t_hbm, x_hbm, lens_hbm, b_hbm, out_hbm,
                  t_v, x_v, lens_v, b_v, out_v, sem_t, sem_x, sem_l, sem_b):
    c = lax.axis_index("c")
    s = lax.axis_index("s")
    wid = s * _NC + c
    base = wid * _RPW
    ct = pltpu.async_copy(t_hbm, t_v, sem_t)
    cx = pltpu.async_copy(x_hbm.at[:, pl.ds(base, _RPW)], x_v, sem_x)
    cl = pltpu.async_copy(lens_hbm.at[pl.ds(base, _RPW)], lens_v, sem_l)
    cb = pltpu.async_copy(b_hbm, b_v, sem_b)
    cx.wait()
    cl.wait()
    cb.wait()
    ct.wait()
    bvec = b_v[...]
    lens_g = [lens_v[pl.ds(g * 16, 16)] for g in range(_NG)]

    def step(j, accs):
        new = []
        for g in range(_NG):
            xi = x_v[j, pl.ds(g * 16, 16)]
            pair = plsc.load_gather(t_v, [xi >> 1])
            bits = (pair >> ((xi & 1) << 4)) << 16
            vals = plsc.bitcast(bits, jnp.float32)
            mask = j < lens_g[g]
            new.append(accs[g] + jnp.where(mask, vals, 0.0))
        return tuple(new)

    def body(i, accs):
        return step(i * 2 + 1, step(i * 2, accs))

    accs = lax.fori_loop(
        0, _L // 2, body,
        tuple(jnp.zeros((16,), jnp.float32) for _ in range(_NG)))
    for g in range(_NG):
        score = accs[g] / lens_g[g].astype(jnp.float32) + bvec
        out_v[pl.ds(g * 16, 16)] = 1.0 / (1.0 + jnp.exp(-score))
    pltpu.sync_copy(out_v, out_hbm.at[pl.ds(base, _RPW)])


_sc_pool = pl.kernel(
    _sc_pool_body,
    out_type=jax.ShapeDtypeStruct((_B,), jnp.float32),
    mesh=plsc.VectorSubcoreMesh(core_axis_name="c", subcore_axis_name="s",
                                num_cores=_NC, num_subcores=_NS),
    compiler_params=pltpu.CompilerParams(needs_layout_passes=False),
    scratch_types=[
        pltpu.VMEM((_TR * 64,), jnp.int32),
        pltpu.VMEM((_L, _RPW), jnp.int32),
        pltpu.VMEM((_RPW,), jnp.int32),
        pltpu.VMEM((16,), jnp.float32),
        pltpu.VMEM((_RPW,), jnp.float32),
        pltpu.SemaphoreType.DMA,
        pltpu.SemaphoreType.DMA,
        pltpu.SemaphoreType.DMA,
        pltpu.SemaphoreType.DMA,
    ],
)


def kernel(X, lens, emb_table, W, b):
    t_bf = _tc_matvec(emb_table.T, W).reshape(_TR * 64, 2)
    t2 = lax.bitcast_convert_type(t_bf, jnp.int32)
    xt = X.astype(jnp.int32).T
    lens_i = lens.astype(jnp.int32)
    b16 = jnp.broadcast_to(b.astype(jnp.float32), (16,))
    probs = _sc_pool(t2, xt, lens_i, b16)
    return probs.reshape(_B, 1)


# speedup vs baseline: 2.1996x; 2.1996x over previous
import jax
import jax.numpy as jnp
from jax import lax
from jax.experimental import pallas as pl
from jax.experimental.pallas import tpu as pltpu
from jax.experimental.pallas import tpu_sc as plsc

_B = 4096
_L = 50
_V = 100000
_D = 64
_NC = 2
_NS = 16
_NW = _NC * _NS
_RPW = _B // _NW
_NG = _RPW // 16
_VB = 7168
_NVB = 7
_SP = _NVB * _VB
_TR = _SP // 128


def _tc_matvec_body(xt_lo_ref, xt_hi_ref, w_ref, o_ref):
    w = w_ref[...]
    olo = lax.dot_general(w, xt_lo_ref[...], (((1,), (0,)), ((), ())),
                          preferred_element_type=jnp.float32)
    ohi = lax.dot_general(w, xt_hi_ref[...], (((1,), (0,)), ((), ())),
                          preferred_element_type=jnp.float32)
    lo = lax.bitcast_convert_type(olo.astype(jnp.bfloat16),
                                  jnp.uint16).astype(jnp.uint32)
    hi = lax.bitcast_convert_type(ohi.astype(jnp.bfloat16),
                                  jnp.uint16).astype(jnp.uint32)
    packed = lax.bitcast_convert_type(lo | (hi << 16), jnp.int32)
    o_ref[...] = packed.reshape(_VB // 128, 128)


def _tc_matvec(emb_t, W):
    return pl.pallas_call(
        _tc_matvec_body,
        grid=(_NVB,),
        in_specs=[
            pl.BlockSpec((_D, _VB), lambda i: (0, i)),
            pl.BlockSpec((_D, _VB), lambda i: (0, i + _NVB)),
            pl.BlockSpec((1, _D), lambda i: (0, 0)),
        ],
        out_specs=pl.BlockSpec((_VB // 128, 128), lambda i: (i, 0)),
        out_shape=jax.ShapeDtypeStruct((_TR, 128), jnp.int32),
    )(emb_t, emb_t, W)


def _sc_pool_body(t_hbm, x_hbm, lens_hbm, b_hbm, out_hbm,
                  t_v, x_v, lens_v, b_v, out_v, sem_t, sem_x, sem_l, sem_b):
    c = lax.axis_index("c")
    s = lax.axis_index("s")
    wid = s * _NC + c
    base = wid * _RPW
    ct = pltpu.async_copy(t_hbm, t_v, sem_t)
    cx = pltpu.async_copy(x_hbm.at[:, pl.ds(base, _RPW)], x_v, sem_x)
    cl = pltpu.async_copy(lens_hbm.at[pl.ds(base, _RPW)], lens_v, sem_l)
    cb = pltpu.async_copy(b_hbm, b_v, sem_b)
    cx.wait()
    cl.wait()
    cb.wait()
    ct.wait()
    bvec = b_v[...]
    lens_g = [lens_v[pl.ds(g * 16, 16)] for g in range(_NG)]

    def step(j, accs):
        new = []
        for g in range(_NG):
            xi = x_v[j, pl.ds(g * 16, 16)]
            ge = xi >= _SP
            pair = plsc.load_gather(t_v, [xi - jnp.where(ge, _SP, 0)])
            bits = (pair >> jnp.where(ge, 16, 0)) << 16
            vals = plsc.bitcast(bits, jnp.float32)
            mask = j < lens_g[g]
            new.append(accs[g] + jnp.where(mask, vals, 0.0))
        return tuple(new)

    def body(i, accs):
        return step(i * 2 + 1, step(i * 2, accs))

    accs = lax.fori_loop(
        0, _L // 2, body,
        tuple(jnp.zeros((16,), jnp.float32) for _ in range(_NG)))
    for g in range(_NG):
        score = accs[g] / lens_g[g].astype(jnp.float32) + bvec
        out_v[pl.ds(g * 16, 16)] = 1.0 / (1.0 + jnp.exp(-score))
    pltpu.sync_copy(out_v, out_hbm.at[pl.ds(base, _RPW)])


_sc_pool = pl.kernel(
    _sc_pool_body,
    out_type=jax.ShapeDtypeStruct((_B,), jnp.float32),
    mesh=plsc.VectorSubcoreMesh(core_axis_name="c", subcore_axis_name="s",
                                num_cores=_NC, num_subcores=_NS),
    compiler_params=pltpu.CompilerParams(needs_layout_passes=False),
    scratch_types=[
        pltpu.VMEM((_SP,), jnp.int32),
        pltpu.VMEM((_L, _RPW), jnp.int32),
        pltpu.VMEM((_RPW,), jnp.int32),
        pltpu.VMEM((16,), jnp.float32),
        pltpu.VMEM((_RPW,), jnp.float32),
        pltpu.SemaphoreType.DMA,
        pltpu.SemaphoreType.DMA,
        pltpu.SemaphoreType.DMA,
        pltpu.SemaphoreType.DMA,
    ],
)


def kernel(X, lens, emb_table, W, b):
    t2 = _tc_matvec(emb_table.T, W).reshape(_SP)
    xt = X.astype(jnp.int32).T
    lens_i = lens.astype(jnp.int32)
    b16 = jnp.broadcast_to(b.astype(jnp.float32), (16,))
    probs = _sc_pool(t2, xt, lens_i, b16)
    return probs.reshape(_B, 1)
